# Initial kernel scaffold; baseline (speedup 1.0000x reference)
import jax
import jax.numpy as jnp
from jax.experimental import pallas as pl


def kernel(x, edge_index, batch, W1_0, b1_0, W2_0, b2_0, W1_1, b1_1, W2_1, b2_1, W1_2, b1_2, W2_2, b2_2, Wp1, bp1, Wp2, bp2):
    raise NotImplementedError("write your pallas kernel here")


# speedup vs baseline: 7.8283x; 7.8283x over previous
import functools

import jax
import jax.numpy as jnp
from jax import lax
from jax.experimental import pallas as pl
from jax.experimental.pallas import tpu as pltpu
from jax.experimental.pallas import tpu_sc as plsc

N = 10000
D = 128
E = 320000
G = 64
OUT = 10

NPAD = 10240
BLK = 1024
NB = NPAD // BLK
CH = 128
NW = 32
K = -(-E // (NW * CH))
EPAD = NW * K * CH
RPT = NPAD // 16


def _dot(a, b):
    return lax.dot_general(a, b, (((1,), (0,)), ((), ())),
                           precision=lax.Precision.HIGHEST,
                           preferred_element_type=jnp.float32)


@functools.lru_cache(maxsize=None)
def _build_sc_agg():
    mesh = plsc.VectorSubcoreMesh(core_axis_name="c", subcore_axis_name="s")

    @functools.partial(
        pl.kernel,
        mesh=mesh,
        out_type=jax.ShapeDtypeStruct((2, NPAD, D), jnp.float32),
        scratch_types=[
            pltpu.VMEM((K, CH), jnp.int32),
            pltpu.VMEM((K, CH), jnp.int32),
            pltpu.VMEM((CH, D), jnp.float32),
            pltpu.VMEM_SHARED((NPAD, D), jnp.float32),
            pltpu.SemaphoreType.DMA,
        ],
    )
    def sc_agg(h_hbm, src_hbm, dst_hbm, zeros_hbm, out_hbm,
               src_v, dst_v, rows_v, agg_sh, sem):
        c = lax.axis_index("c")
        s = lax.axis_index("s")
        wid = s * 2 + c
        pltpu.sync_copy(zeros_hbm, agg_sh.at[pl.ds(s * RPT, RPT)])
        plsc.subcore_barrier()
        pltpu.sync_copy(src_hbm.at[wid], src_v)
        pltpu.sync_copy(dst_hbm.at[wid], dst_v)

        def body(ci, carry):
            pltpu.async_copy(h_hbm.at[src_v.at[ci]], rows_v, sem).wait()
            pltpu.sync_copy(rows_v, agg_sh.at[dst_v.at[ci]], add=True)
            return carry

        lax.fori_loop(0, K, body, 0)
        plsc.subcore_barrier()
        pltpu.sync_copy(agg_sh.at[pl.ds(s * RPT, RPT)],
                        out_hbm.at[c, pl.ds(s * RPT, RPT)])

    return sc_agg


def _mlp_body(x_ref, a0_ref, a1_ref, w1_ref, b1_ref, w2_ref, b2_ref,
              pre_ref, post_ref):
    h = x_ref[...] + a0_ref[0] + a1_ref[0]
    t = jnp.maximum(_dot(h, w1_ref[...]) + b1_ref[...], 0.0)
    o = _dot(t, w2_ref[...]) + b2_ref[...]
    pre_ref[...] = o
    post_ref[...] = jnp.maximum(o, 0.0)


def _mlp(xp, aggs, W1, b1, W2, b2):
    return pl.pallas_call(
        _mlp_body,
        grid=(NB,),
        in_specs=[
            pl.BlockSpec((BLK, D), lambda i: (i, 0)),
            pl.BlockSpec((1, BLK, D), lambda i: (0, i, 0)),
            pl.BlockSpec((1, BLK, D), lambda i: (1, i, 0)),
            pl.BlockSpec((D, D), lambda i: (0, 0)),
            pl.BlockSpec((1, D), lambda i: (0, 0)),
            pl.BlockSpec((D, D), lambda i: (0, 0)),
            pl.BlockSpec((1, D), lambda i: (0, 0)),
        ],
        out_specs=[pl.BlockSpec((BLK, D), lambda i: (i, 0))] * 2,
        out_shape=[jax.ShapeDtypeStruct((NPAD, D), jnp.float32)] * 2,
    )(xp, aggs, aggs, W1, b1.reshape(1, D), W2, b2.reshape(1, D))


def _pool_body(h_ref, b_ref, wp1_ref, bp1_ref, wp2_ref, bp2_ref, out_ref,
               acc, cnt):
    i = pl.program_id(0)

    @pl.when(i == 0)
    def _():
        acc[...] = jnp.zeros_like(acc)
        cnt[...] = jnp.zeros_like(cnt)

    b = b_ref[0]
    gids = lax.broadcasted_iota(jnp.int32, (G, BLK), 0)
    oh = (gids == b).astype(jnp.float32)
    acc[...] += _dot(oh, h_ref[...])
    cnt[...] += _dot(oh, jnp.ones((BLK, D), jnp.float32))

    @pl.when(i == NB - 1)
    def _():
        pooled = acc[...] / jnp.maximum(cnt[...], 1.0)
        o1 = _dot(pooled, wp1_ref[...]) + bp1_ref[...]
        o2 = _dot(o1, wp2_ref[...]) + bp2_ref[...]
        m = jnp.max(o2, axis=1, keepdims=True)
        lse = m + jnp.log(jnp.sum(jnp.exp(o2 - m), axis=1, keepdims=True))
        out_ref[...] = o2 - lse


def _pool(post, batch3, Wp1, bp1, Wp2p, bp2p):
    return pl.pallas_call(
        _pool_body,
        grid=(NB,),
        in_specs=[
            pl.BlockSpec((BLK, D), lambda i: (i, 0)),
            pl.BlockSpec((1, 1, BLK), lambda i: (i, 0, 0)),
            pl.BlockSpec((D, D), lambda i: (0, 0)),
            pl.BlockSpec((1, D), lambda i: (0, 0)),
            pl.BlockSpec((D, D), lambda i: (0, 0)),
            pl.BlockSpec((1, D), lambda i: (0, 0)),
        ],
        out_specs=pl.BlockSpec((G, D), lambda i: (0, 0)),
        out_shape=jax.ShapeDtypeStruct((G, D), jnp.float32),
        scratch_shapes=[pltpu.VMEM((G, D), jnp.float32),
                        pltpu.VMEM((G, D), jnp.float32)],
    )(post, batch3, Wp1, bp1.reshape(1, D), Wp2p, bp2p)


def kernel(x, edge_index, batch,
           W1_0, b1_0, W2_0, b2_0,
           W1_1, b1_1, W2_1, b2_1,
           W1_2, b1_2, W2_2, b2_2,
           Wp1, bp1, Wp2, bp2):
    src = edge_index[0]
    dst = edge_index[1]
    pad_e = EPAD - E
    ar = jnp.arange(pad_e, dtype=jnp.int32)
    pad_src = (ar * 37) % N
    pad_dst = N + ar % (NPAD - N)
    src3 = jnp.concatenate([src, pad_src]).reshape(NW, K, CH)
    dst3 = jnp.concatenate([dst, pad_dst]).reshape(NW, K, CH)
    zeros = jnp.zeros((RPT, D), jnp.float32)

    xp = jnp.pad(x, ((0, NPAD - N), (0, 0)))
    batch3 = jnp.pad(batch, (0, NPAD - N), constant_values=G).reshape(NB, 1, BLK)
    Wp2p = jnp.pad(Wp2, ((0, 0), (0, D - OUT)))
    bp2p = jnp.pad(bp2, (0, D - OUT), constant_values=-1e30).reshape(1, D)

    sc_agg = _build_sc_agg()

    h = xp
    pre = None
    for (W1, b1, W2, b2) in ((W1_0, b1_0, W2_0, b2_0),
                             (W1_1, b1_1, W2_1, b2_1),
                             (W1_2, b1_2, W2_2, b2_2)):
        aggs = sc_agg(h, src3, dst3, zeros)
        pre, post = _mlp(h, aggs, W1, b1, W2, b2)
        h = post

    emb = pre[:N]
    logp = _pool(h, batch3, Wp1, bp1, Wp2p, bp2p)[:, :OUT]
    return emb, logp


# speedup vs baseline: 11.2732x; 1.4401x over previous
import functools

import jax
import jax.numpy as jnp
from jax import lax
from jax.experimental import pallas as pl
from jax.experimental.pallas import tpu as pltpu
from jax.experimental.pallas import tpu_sc as plsc

N = 10000
D = 128
E = 320000
G = 64
OUT = 10

NPAD = 10240
BLK = 1024
NB = NPAD // BLK
CH = 128
NW = 32
NBUF = 2
K = 80
KH = K // 2
EPAD = NW * K * CH
RPT = NPAD // 16


def _dot(a, b):
    return lax.dot_general(a, b, (((1,), (0,)), ((), ())),
                           precision=lax.Precision.HIGHEST,
                           preferred_element_type=jnp.float32)


@functools.lru_cache(maxsize=None)
def _build_sc_agg():
    mesh = plsc.VectorSubcoreMesh(core_axis_name="c", subcore_axis_name="s")

    @functools.partial(
        pl.kernel,
        mesh=mesh,
        out_type=jax.ShapeDtypeStruct((2, NPAD, D), jnp.float32),
        scratch_types=[
            pltpu.VMEM((KH, CH), jnp.int32),
            pltpu.VMEM((KH, CH), jnp.int32),
            pltpu.VMEM((NBUF, CH, D), jnp.float32),
            pltpu.VMEM_SHARED((NPAD, D), jnp.float32),
            pltpu.SemaphoreType.DMA,
            pltpu.SemaphoreType.DMA,
        ],
    )
    def sc_agg(h_hbm, src_hbm, dst_hbm, zeros_hbm, out_hbm,
               src_v, dst_v, rows_v, agg_sh, *sems):
        c = lax.axis_index("c")
        s = lax.axis_index("s")
        wid = s * 2 + c
        pltpu.sync_copy(zeros_hbm, agg_sh.at[pl.ds(s * RPT, RPT)])
        plsc.subcore_barrier()

        for ph in range(K // KH):
            pltpu.sync_copy(src_hbm.at[wid, pl.ds(ph * KH, KH)], src_v)
            pltpu.sync_copy(dst_hbm.at[wid, pl.ds(ph * KH, KH)], dst_v)

            for b in range(NBUF):
                pltpu.async_copy(h_hbm.at[src_v.at[b]], rows_v.at[b], sems[b])

            def body(g, carry):
                for b in range(NBUF):
                    ci = g * NBUF + b
                    pltpu.make_async_copy(h_hbm.at[src_v.at[ci]],
                                          rows_v.at[b], sems[b]).wait()
                    pltpu.sync_copy(rows_v.at[b], agg_sh.at[dst_v.at[ci]],
                                    add=True)

                    @pl.when(ci + NBUF < KH)
                    def _():
                        pltpu.async_copy(h_hbm.at[src_v.at[ci + NBUF]],
                                         rows_v.at[b], sems[b])

                return carry

            lax.fori_loop(0, KH // NBUF, body, 0)

        plsc.subcore_barrier()
        pltpu.sync_copy(agg_sh.at[pl.ds(s * RPT, RPT)],
                        out_hbm.at[c, pl.ds(s * RPT, RPT)])

    return sc_agg


def _mlp_body(x_ref, a0_ref, a1_ref, w1_ref, b1_ref, w2_ref, b2_ref,
              pre_ref, post_ref):
    h = x_ref[...] + a0_ref[0] + a1_ref[0]
    t = jnp.maximum(_dot(h, w1_ref[...]) + b1_ref[...], 0.0)
    o = _dot(t, w2_ref[...]) + b2_ref[...]
    pre_ref[...] = o
    post_ref[...] = jnp.maximum(o, 0.0)


def _mlp(xp, aggs, W1, b1, W2, b2):
    return pl.pallas_call(
        _mlp_body,
        grid=(NB,),
        in_specs=[
            pl.BlockSpec((BLK, D), lambda i: (i, 0)),
            pl.BlockSpec((1, BLK, D), lambda i: (0, i, 0)),
            pl.BlockSpec((1, BLK, D), lambda i: (1, i, 0)),
            pl.BlockSpec((D, D), lambda i: (0, 0)),
            pl.BlockSpec((1, D), lambda i: (0, 0)),
            pl.BlockSpec((D, D), lambda i: (0, 0)),
            pl.BlockSpec((1, D), lambda i: (0, 0)),
        ],
        out_specs=[pl.BlockSpec((BLK, D), lambda i: (i, 0))] * 2,
        out_shape=[jax.ShapeDtypeStruct((NPAD, D), jnp.float32)] * 2,
    )(xp, aggs, aggs, W1, b1.reshape(1, D), W2, b2.reshape(1, D))


def _pool_body(h_ref, b_ref, wp1_ref, bp1_ref, wp2_ref, bp2_ref, out_ref,
               acc, cnt):
    i = pl.program_id(0)

    @pl.when(i == 0)
    def _():
        acc[...] = jnp.zeros_like(acc)
        cnt[...] = jnp.zeros_like(cnt)

    b = b_ref[0]
    gids = lax.broadcasted_iota(jnp.int32, (G, BLK), 0)
    oh = (gids == b).astype(jnp.float32)
    acc[...] += _dot(oh, h_ref[...])
    cnt[...] += _dot(oh, jnp.ones((BLK, D), jnp.float32))

    @pl.when(i == NB - 1)
    def _():
        pooled = acc[...] / jnp.maximum(cnt[...], 1.0)
        o1 = _dot(pooled, wp1_ref[...]) + bp1_ref[...]
        o2 = _dot(o1, wp2_ref[...]) + bp2_ref[...]
        m = jnp.max(o2, axis=1, keepdims=True)
        lse = m + jnp.log(jnp.sum(jnp.exp(o2 - m), axis=1, keepdims=True))
        out_ref[...] = o2 - lse


def _pool(post, batch3, Wp1, bp1, Wp2p, bp2p):
    return pl.pallas_call(
        _pool_body,
        grid=(NB,),
        in_specs=[
            pl.BlockSpec((BLK, D), lambda i: (i, 0)),
            pl.BlockSpec((1, 1, BLK), lambda i: (i, 0, 0)),
            pl.BlockSpec((D, D), lambda i: (0, 0)),
            pl.BlockSpec((1, D), lambda i: (0, 0)),
            pl.BlockSpec((D, D), lambda i: (0, 0)),
            pl.BlockSpec((1, D), lambda i: (0, 0)),
        ],
        out_specs=pl.BlockSpec((G, D), lambda i: (0, 0)),
        out_shape=jax.ShapeDtypeStruct((G, D), jnp.float32),
        scratch_shapes=[pltpu.VMEM((G, D), jnp.float32),
                        pltpu.VMEM((G, D), jnp.float32)],
    )(post, batch3, Wp1, bp1.reshape(1, D), Wp2p, bp2p)


def kernel(x, edge_index, batch,
           W1_0, b1_0, W2_0, b2_0,
           W1_1, b1_1, W2_1, b2_1,
           W1_2, b1_2, W2_2, b2_2,
           Wp1, bp1, Wp2, bp2):
    src = edge_index[0]
    dst = edge_index[1]
    pad_e = EPAD - E
    ar = jnp.arange(pad_e, dtype=jnp.int32)
    pad_src = (ar * 37) % N
    pad_dst = N + ar % (NPAD - N)
    src3 = jnp.concatenate([src, pad_src]).reshape(NW, K, CH)
    dst3 = jnp.concatenate([dst, pad_dst]).reshape(NW, K, CH)
    zeros = jnp.zeros((RPT, D), jnp.float32)

    xp = jnp.pad(x, ((0, NPAD - N), (0, 0)))
    batch3 = jnp.pad(batch, (0, NPAD - N), constant_values=G).reshape(NB, 1, BLK)
    Wp2p = jnp.pad(Wp2, ((0, 0), (0, D - OUT)))
    bp2p = jnp.pad(bp2, (0, D - OUT), constant_values=-1e30).reshape(1, D)

    sc_agg = _build_sc_agg()

    h = xp
    pre = None
    for (W1, b1, W2, b2) in ((W1_0, b1_0, W2_0, b2_0),
                             (W1_1, b1_1, W2_1, b2_1),
                             (W1_2, b1_2, W2_2, b2_2)):
        aggs = sc_agg(h, src3, dst3, zeros)
        pre, post = _mlp(h, aggs, W1, b1, W2, b2)
        h = post

    emb = pre[:N]
    logp = _pool(h, batch3, Wp1, bp1, Wp2p, bp2p)[:, :OUT]
    return emb, logp


# speedup vs baseline: 11.5046x; 1.0205x over previous
import functools

import jax
import jax.numpy as jnp
from jax import lax
from jax.experimental import pallas as pl
from jax.experimental.pallas import tpu as pltpu
from jax.experimental.pallas import tpu_sc as plsc

N = 10000
D = 128
E = 320000
G = 64
OUT = 10

NPAD = 10240
BLK = 1024
NB = NPAD // BLK
CH = 128
NW = 32
NBUF = 2
K = 80
KH = K // 2
EPAD = NW * K * CH
RPT = NPAD // 16


def _dot(a, b):
    return lax.dot_general(a, b, (((1,), (0,)), ((), ())),
                           precision=lax.Precision.HIGHEST,
                           preferred_element_type=jnp.float32)


@functools.lru_cache(maxsize=None)
def _build_sc_agg():
    mesh = plsc.VectorSubcoreMesh(core_axis_name="c", subcore_axis_name="s")

    @functools.partial(
        pl.kernel,
        mesh=mesh,
        out_type=jax.ShapeDtypeStruct((2, NPAD, D), jnp.float32),
        scratch_types=[
            pltpu.VMEM((KH, CH), jnp.int32),
            pltpu.VMEM((KH, CH), jnp.int32),
            pltpu.VMEM((NBUF, CH, D), jnp.float32),
            pltpu.VMEM_SHARED((NPAD, D), jnp.float32),
            pltpu.SemaphoreType.DMA,
            pltpu.SemaphoreType.DMA,
        ],
    )
    def sc_agg(h_hbm, src_hbm, dst_hbm, zeros_hbm, out_hbm,
               src_v, dst_v, rows_v, agg_sh, *sems):
        c = lax.axis_index("c")
        s = lax.axis_index("s")
        wid = s * 2 + c

        @pl.when(c == 0)
        def _():
            pltpu.sync_copy(h_hbm.at[pl.ds(s * RPT, RPT)],
                            agg_sh.at[pl.ds(s * RPT, RPT)])

        @pl.when(c != 0)
        def _():
            pltpu.sync_copy(zeros_hbm, agg_sh.at[pl.ds(s * RPT, RPT)])

        plsc.subcore_barrier()

        for ph in range(K // KH):
            pltpu.sync_copy(src_hbm.at[wid, pl.ds(ph * KH, KH)], src_v)
            pltpu.sync_copy(dst_hbm.at[wid, pl.ds(ph * KH, KH)], dst_v)

            for b in range(NBUF):
                pltpu.async_copy(h_hbm.at[src_v.at[b]], rows_v.at[b], sems[b])

            def body(g, carry):
                for b in range(NBUF):
                    ci = g * NBUF + b
                    pltpu.make_async_copy(h_hbm.at[src_v.at[ci]],
                                          rows_v.at[b], sems[b]).wait()
                    pltpu.sync_copy(rows_v.at[b], agg_sh.at[dst_v.at[ci]],
                                    add=True)

                    @pl.when(ci + NBUF < KH)
                    def _():
                        pltpu.async_copy(h_hbm.at[src_v.at[ci + NBUF]],
                                         rows_v.at[b], sems[b])

                return carry

            lax.fori_loop(0, KH // NBUF, body, 0)

        plsc.subcore_barrier()
        pltpu.sync_copy(agg_sh.at[pl.ds(s * RPT, RPT)],
                        out_hbm.at[c, pl.ds(s * RPT, RPT)])

    return sc_agg


def _mlp_body(a0_ref, a1_ref, w1_ref, b1_ref, w2_ref, b2_ref, post_ref):
    h = a0_ref[0] + a1_ref[0]
    t = jnp.maximum(_dot(h, w1_ref[...]) + b1_ref[...], 0.0)
    o = _dot(t, w2_ref[...]) + b2_ref[...]
    post_ref[...] = jnp.maximum(o, 0.0)


_W_SPECS = [
    pl.BlockSpec((D, D), lambda i: (0, 0)),
    pl.BlockSpec((1, D), lambda i: (0, 0)),
    pl.BlockSpec((D, D), lambda i: (0, 0)),
    pl.BlockSpec((1, D), lambda i: (0, 0)),
]


def _mlp(aggs, W1, b1, W2, b2):
    return pl.pallas_call(
        _mlp_body,
        grid=(NB,),
        in_specs=[
            pl.BlockSpec((1, BLK, D), lambda i: (0, i, 0)),
            pl.BlockSpec((1, BLK, D), lambda i: (1, i, 0)),
        ] + _W_SPECS,
        out_specs=pl.BlockSpec((BLK, D), lambda i: (i, 0)),
        out_shape=jax.ShapeDtypeStruct((NPAD, D), jnp.float32),
    )(aggs, aggs, W1, b1.reshape(1, D), W2, b2.reshape(1, D))


def _mlp_pool_body(a0_ref, a1_ref, w1_ref, b1_ref, w2_ref, b2_ref,
                   b3_ref, wp1_ref, bp1_ref, wp2_ref, bp2_ref,
                   emb_ref, out_ref, acc, cnt):
    i = pl.program_id(0)

    @pl.when(i == 0)
    def _():
        acc[...] = jnp.zeros_like(acc)
        cnt[...] = jnp.zeros_like(cnt)

    h = a0_ref[0] + a1_ref[0]
    t = jnp.maximum(_dot(h, w1_ref[...]) + b1_ref[...], 0.0)
    o = _dot(t, w2_ref[...]) + b2_ref[...]
    emb_ref[...] = o

    hp = jnp.maximum(o, 0.0)
    b = b3_ref[0]
    gids = lax.broadcasted_iota(jnp.int32, (G, BLK), 0)
    oh = (gids == b).astype(jnp.float32)
    acc[...] += _dot(oh, hp)
    cnt[...] += _dot(oh, jnp.ones((BLK, D), jnp.float32))

    @pl.when(i == NB - 1)
    def _():
        pooled = acc[...] / jnp.maximum(cnt[...], 1.0)
        o1 = _dot(pooled, wp1_ref[...]) + bp1_ref[...]
        o2 = _dot(o1, wp2_ref[...]) + bp2_ref[...]
        m = jnp.max(o2, axis=1, keepdims=True)
        lse = m + jnp.log(jnp.sum(jnp.exp(o2 - m), axis=1, keepdims=True))
        out_ref[...] = o2 - lse


def _mlp_pool(aggs, W1, b1, W2, b2, batch3, Wp1, bp1, Wp2p, bp2p):
    return pl.pallas_call(
        _mlp_pool_body,
        grid=(NB,),
        in_specs=[
            pl.BlockSpec((1, BLK, D), lambda i: (0, i, 0)),
            pl.BlockSpec((1, BLK, D), lambda i: (1, i, 0)),
        ] + _W_SPECS + [
            pl.BlockSpec((1, 1, BLK), lambda i: (i, 0, 0)),
        ] + _W_SPECS,
        out_specs=[pl.BlockSpec((BLK, D), lambda i: (i, 0)),
                   pl.BlockSpec((G, D), lambda i: (0, 0))],
        out_shape=[jax.ShapeDtypeStruct((NPAD, D), jnp.float32),
                   jax.ShapeDtypeStruct((G, D), jnp.float32)],
        scratch_shapes=[pltpu.VMEM((G, D), jnp.float32),
                        pltpu.VMEM((G, D), jnp.float32)],
    )(aggs, aggs, W1, b1.reshape(1, D), W2, b2.reshape(1, D),
      batch3, Wp1, bp1.reshape(1, D), Wp2p, bp2p)


def kernel(x, edge_index, batch,
           W1_0, b1_0, W2_0, b2_0,
           W1_1, b1_1, W2_1, b2_1,
           W1_2, b1_2, W2_2, b2_2,
           Wp1, bp1, Wp2, bp2):
    src = edge_index[0]
    dst = edge_index[1]
    pad_e = EPAD - E
    ar = jnp.arange(pad_e, dtype=jnp.int32)
    pad_src = (ar * 37) % N
    pad_dst = N + ar % (NPAD - N)
    src3 = jnp.concatenate([src, pad_src]).reshape(NW, K, CH)
    dst3 = jnp.concatenate([dst, pad_dst]).reshape(NW, K, CH)
    zeros = jnp.zeros((RPT, D), jnp.float32)

    xp = jnp.pad(x, ((0, NPAD - N), (0, 0)))
    batch3 = jnp.pad(batch, (0, NPAD - N), constant_values=G).reshape(NB, 1, BLK)
    Wp2p = jnp.pad(Wp2, ((0, 0), (0, D - OUT)))
    bp2p = jnp.pad(bp2, (0, D - OUT), constant_values=-1e30).reshape(1, D)

    sc_agg = _build_sc_agg()

    h = xp
    for (W1, b1, W2, b2) in ((W1_0, b1_0, W2_0, b2_0),
                             (W1_1, b1_1, W2_1, b2_1)):
        aggs = sc_agg(h, src3, dst3, zeros)
        h = _mlp(aggs, W1, b1, W2, b2)

    aggs = sc_agg(h, src3, dst3, zeros)
    emb_p, logp_p = _mlp_pool(aggs, W1_2, b1_2, W2_2, b2_2,
                              batch3, Wp1, bp1, Wp2p, bp2p)
    return emb_p[:N], logp_p[:, :OUT]


# speedup vs baseline: 11.8773x; 1.0324x over previous
import functools

import jax
import jax.numpy as jnp
from jax import lax
from jax.experimental import pallas as pl
from jax.experimental.pallas import tpu as pltpu
from jax.experimental.pallas import tpu_sc as plsc

N = 10000
D = 128
E = 320000
G = 64
OUT = 10

NPAD = 10240
BLK = 2048
NB = NPAD // BLK
CH = 128
NW = 32
NBUF = 2
K = 80
KH = K // 2
EPAD = NW * K * CH
RPT = NPAD // 16


def _dot(a, b):
    return lax.dot_general(a, b, (((1,), (0,)), ((), ())),
                           precision=lax.Precision.HIGHEST,
                           preferred_element_type=jnp.float32)


@functools.lru_cache(maxsize=None)
def _build_sc_agg():
    mesh = plsc.VectorSubcoreMesh(core_axis_name="c", subcore_axis_name="s")

    @functools.partial(
        pl.kernel,
        mesh=mesh,
        out_type=jax.ShapeDtypeStruct((2, NPAD, D), jnp.float32),
        scratch_types=[
            pltpu.VMEM((KH, CH), jnp.int32),
            pltpu.VMEM((KH, CH), jnp.int32),
            pltpu.VMEM((NBUF, CH, D), jnp.float32),
            pltpu.VMEM_SHARED((NPAD, D), jnp.float32),
            pltpu.SemaphoreType.DMA,
            pltpu.SemaphoreType.DMA,
        ],
    )
    def sc_agg(h_hbm, src_hbm, dst_hbm, zeros_hbm, out_hbm,
               src_v, dst_v, rows_v, agg_sh, *sems):
        c = lax.axis_index("c")
        s = lax.axis_index("s")
        wid = s * 2 + c

        for ph in range(K // KH):
            pltpu.sync_copy(src_hbm.at[wid, pl.ds(ph * KH, KH)], src_v)
            pltpu.sync_copy(dst_hbm.at[wid, pl.ds(ph * KH, KH)], dst_v)

            for b in range(NBUF):
                pltpu.async_copy(h_hbm.at[src_v.at[b]], rows_v.at[b], sems[b])

            if ph == 0:
                @pl.when(c == 0)
                def _():
                    pltpu.sync_copy(h_hbm.at[pl.ds(s * RPT, RPT)],
                                    agg_sh.at[pl.ds(s * RPT, RPT)])

                @pl.when(c != 0)
                def _():
                    pltpu.sync_copy(zeros_hbm, agg_sh.at[pl.ds(s * RPT, RPT)])

                plsc.subcore_barrier()

            def body(g, carry):
                for b in range(NBUF):
                    ci = g * NBUF + b
                    pltpu.make_async_copy(h_hbm.at[src_v.at[ci]],
                                          rows_v.at[b], sems[b]).wait()
                    pltpu.sync_copy(rows_v.at[b], agg_sh.at[dst_v.at[ci]],
                                    add=True)

                    @pl.when(ci + NBUF < KH)
                    def _():
                        pltpu.async_copy(h_hbm.at[src_v.at[ci + NBUF]],
                                         rows_v.at[b], sems[b])

                return carry

            lax.fori_loop(0, KH // NBUF, body, 0)

        plsc.subcore_barrier()
        pltpu.sync_copy(agg_sh.at[pl.ds(s * RPT, RPT)],
                        out_hbm.at[c, pl.ds(s * RPT, RPT)])

    return sc_agg


def _mlp_body(a0_ref, a1_ref, w1_ref, b1_ref, w2_ref, b2_ref, post_ref):
    h = a0_ref[0] + a1_ref[0]
    t = jnp.maximum(_dot(h, w1_ref[...]) + b1_ref[...], 0.0)
    o = _dot(t, w2_ref[...]) + b2_ref[...]
    post_ref[...] = jnp.maximum(o, 0.0)


_W_SPECS = [
    pl.BlockSpec((D, D), lambda i: (0, 0)),
    pl.BlockSpec((1, D), lambda i: (0, 0)),
    pl.BlockSpec((D, D), lambda i: (0, 0)),
    pl.BlockSpec((1, D), lambda i: (0, 0)),
]


def _mlp(aggs, W1, b1, W2, b2):
    return pl.pallas_call(
        _mlp_body,
        grid=(NB,),
        in_specs=[
            pl.BlockSpec((1, BLK, D), lambda i: (0, i, 0)),
            pl.BlockSpec((1, BLK, D), lambda i: (1, i, 0)),
        ] + _W_SPECS,
        out_specs=pl.BlockSpec((BLK, D), lambda i: (i, 0)),
        out_shape=jax.ShapeDtypeStruct((NPAD, D), jnp.float32),
    )(aggs, aggs, W1, b1.reshape(1, D), W2, b2.reshape(1, D))


def _mlp_pool_body(a0_ref, a1_ref, w1_ref, b1_ref, w2_ref, b2_ref,
                   b3_ref, wp1_ref, bp1_ref, wp2_ref, bp2_ref,
                   emb_ref, out_ref, acc, cnt):
    i = pl.program_id(0)

    @pl.when(i == 0)
    def _():
        acc[...] = jnp.zeros_like(acc)
        cnt[...] = jnp.zeros_like(cnt)

    h = a0_ref[0] + a1_ref[0]
    t = jnp.maximum(_dot(h, w1_ref[...]) + b1_ref[...], 0.0)
    o = _dot(t, w2_ref[...]) + b2_ref[...]
    emb_ref[...] = o

    hp = jnp.maximum(o, 0.0)
    b = b3_ref[0]
    gids = lax.broadcasted_iota(jnp.int32, (G, BLK), 0)
    oh = (gids == b).astype(jnp.float32)
    acc[...] += _dot(oh, hp)
    cnt[...] += _dot(oh, jnp.ones((BLK, D), jnp.float32))

    @pl.when(i == NB - 1)
    def _():
        pooled = acc[...] / jnp.maximum(cnt[...], 1.0)
        o1 = _dot(pooled, wp1_ref[...]) + bp1_ref[...]
        o2 = _dot(o1, wp2_ref[...]) + bp2_ref[...]
        m = jnp.max(o2, axis=1, keepdims=True)
        lse = m + jnp.log(jnp.sum(jnp.exp(o2 - m), axis=1, keepdims=True))
        out_ref[...] = o2 - lse


def _mlp_pool(aggs, W1, b1, W2, b2, batch3, Wp1, bp1, Wp2p, bp2p):
    return pl.pallas_call(
        _mlp_pool_body,
        grid=(NB,),
        in_specs=[
            pl.BlockSpec((1, BLK, D), lambda i: (0, i, 0)),
            pl.BlockSpec((1, BLK, D), lambda i: (1, i, 0)),
        ] + _W_SPECS + [
            pl.BlockSpec((1, 1, BLK), lambda i: (i, 0, 0)),
        ] + _W_SPECS,
        out_specs=[pl.BlockSpec((BLK, D), lambda i: (i, 0)),
                   pl.BlockSpec((G, D), lambda i: (0, 0))],
        out_shape=[jax.ShapeDtypeStruct((N, D), jnp.float32),
                   jax.ShapeDtypeStruct((G, D), jnp.float32)],
        scratch_shapes=[pltpu.VMEM((G, D), jnp.float32),
                        pltpu.VMEM((G, D), jnp.float32)],
    )(aggs, aggs, W1, b1.reshape(1, D), W2, b2.reshape(1, D),
      batch3, Wp1, bp1.reshape(1, D), Wp2p, bp2p)


def kernel(x, edge_index, batch,
           W1_0, b1_0, W2_0, b2_0,
           W1_1, b1_1, W2_1, b2_1,
           W1_2, b1_2, W2_2, b2_2,
           Wp1, bp1, Wp2, bp2):
    src = edge_index[0]
    dst = edge_index[1]
    pad_e = EPAD - E
    ar = jnp.arange(pad_e, dtype=jnp.int32)
    pad_src = (ar * 37) % N
    pad_dst = N + ar % (NPAD - N)
    src3 = jnp.concatenate([src, pad_src]).reshape(NW, K, CH)
    dst3 = jnp.concatenate([dst, pad_dst]).reshape(NW, K, CH)
    zeros = jnp.zeros((RPT, D), jnp.float32)

    xp = jnp.pad(x, ((0, NPAD - N), (0, 0)))
    batch3 = jnp.pad(batch, (0, NPAD - N), constant_values=G).reshape(NB, 1, BLK)
    Wp2p = jnp.pad(Wp2, ((0, 0), (0, D - OUT)))
    bp2p = jnp.pad(bp2, (0, D - OUT), constant_values=-1e30).reshape(1, D)

    sc_agg = _build_sc_agg()

    h = xp
    for (W1, b1, W2, b2) in ((W1_0, b1_0, W2_0, b2_0),
                             (W1_1, b1_1, W2_1, b2_1)):
        aggs = sc_agg(h, src3, dst3, zeros)
        h = _mlp(aggs, W1, b1, W2, b2)

    aggs = sc_agg(h, src3, dst3, zeros)
    emb, logp_p = _mlp_pool(aggs, W1_2, b1_2, W2_2, b2_2,
                            batch3, Wp1, bp1, Wp2p, bp2p)
    return emb, logp_p[:, :OUT]


# speedup vs baseline: 12.9893x; 1.0936x over previous
import functools

import jax
import jax.numpy as jnp
from jax import lax
from jax.experimental import pallas as pl
from jax.experimental.pallas import tpu as pltpu
from jax.experimental.pallas import tpu_sc as plsc

N = 10000
D = 128
E = 320000
G = 64
OUT = 10

NPAD = 10240
BLK = 2048
NB = NPAD // BLK
CH = 128
NW = 32
NBUF = 2
K = 80
KH = K // 2
EPAD = NW * K * CH
RPT = NPAD // 16


def _dot(a, b, precision=lax.Precision.DEFAULT):
    return lax.dot_general(a, b, (((1,), (0,)), ((), ())),
                           precision=precision,
                           preferred_element_type=jnp.float32)


@functools.lru_cache(maxsize=None)
def _build_sc_agg():
    mesh = plsc.VectorSubcoreMesh(core_axis_name="c", subcore_axis_name="s")

    @functools.partial(
        pl.kernel,
        mesh=mesh,
        out_type=jax.ShapeDtypeStruct((2, NPAD, D), jnp.float32),
        scratch_types=[
            pltpu.VMEM((KH, CH), jnp.int32),
            pltpu.VMEM((KH, CH), jnp.int32),
            pltpu.VMEM((NBUF, CH, D), jnp.float32),
            pltpu.VMEM_SHARED((NPAD, D), jnp.float32),
            pltpu.SemaphoreType.DMA,
            pltpu.SemaphoreType.DMA,
        ],
    )
    def sc_agg(h_hbm, src_hbm, dst_hbm, zeros_hbm, out_hbm,
               src_v, dst_v, rows_v, agg_sh, *sems):
        c = lax.axis_index("c")
        s = lax.axis_index("s")
        wid = s * 2 + c

        for ph in range(K // KH):
            pltpu.sync_copy(src_hbm.at[wid, pl.ds(ph * KH, KH)], src_v)
            pltpu.sync_copy(dst_hbm.at[wid, pl.ds(ph * KH, KH)], dst_v)

            for b in range(NBUF):
                pltpu.async_copy(h_hbm.at[src_v.at[b]], rows_v.at[b], sems[b])

            if ph == 0:
                @pl.when(c == 0)
                def _():
                    pltpu.sync_copy(h_hbm.at[pl.ds(s * RPT, RPT)],
                                    agg_sh.at[pl.ds(s * RPT, RPT)])

                @pl.when(c != 0)
                def _():
                    pltpu.sync_copy(zeros_hbm, agg_sh.at[pl.ds(s * RPT, RPT)])

                plsc.subcore_barrier()

            def body(g, carry):
                for b in range(NBUF):
                    ci = g * NBUF + b
                    pltpu.make_async_copy(h_hbm.at[src_v.at[ci]],
                                          rows_v.at[b], sems[b]).wait()
                    pltpu.sync_copy(rows_v.at[b], agg_sh.at[dst_v.at[ci]],
                                    add=True)

                    @pl.when(ci + NBUF < KH)
                    def _():
                        pltpu.async_copy(h_hbm.at[src_v.at[ci + NBUF]],
                                         rows_v.at[b], sems[b])

                return carry

            lax.fori_loop(0, KH // NBUF, body, 0)

        plsc.subcore_barrier()
        pltpu.sync_copy(agg_sh.at[pl.ds(s * RPT, RPT)],
                        out_hbm.at[c, pl.ds(s * RPT, RPT)])

    return sc_agg


def _mlp_body(a0_ref, a1_ref, w1_ref, b1_ref, w2_ref, b2_ref, post_ref):
    h = a0_ref[0] + a1_ref[0]
    t = jnp.maximum(_dot(h, w1_ref[...]) + b1_ref[...], 0.0)
    o = _dot(t, w2_ref[...]) + b2_ref[...]
    post_ref[...] = jnp.maximum(o, 0.0)


_W_SPECS = [
    pl.BlockSpec((D, D), lambda i: (0, 0)),
    pl.BlockSpec((1, D), lambda i: (0, 0)),
    pl.BlockSpec((D, D), lambda i: (0, 0)),
    pl.BlockSpec((1, D), lambda i: (0, 0)),
]


def _mlp(aggs, W1, b1, W2, b2):
    return pl.pallas_call(
        _mlp_body,
        grid=(NB,),
        in_specs=[
            pl.BlockSpec((1, BLK, D), lambda i: (0, i, 0)),
            pl.BlockSpec((1, BLK, D), lambda i: (1, i, 0)),
        ] + _W_SPECS,
        out_specs=pl.BlockSpec((BLK, D), lambda i: (i, 0)),
        out_shape=jax.ShapeDtypeStruct((NPAD, D), jnp.float32),
    )(aggs, aggs, W1, b1.reshape(1, D), W2, b2.reshape(1, D))


def _mlp_pool_body(a0_ref, a1_ref, w1_ref, b1_ref, w2_ref, b2_ref,
                   b3_ref, wp1_ref, bp1_ref, wp2_ref, bp2_ref,
                   emb_ref, out_ref, acc, cnt):
    i = pl.program_id(0)

    @pl.when(i == 0)
    def _():
        acc[...] = jnp.zeros_like(acc)
        cnt[...] = jnp.zeros_like(cnt)

    h = a0_ref[0] + a1_ref[0]
    t = jnp.maximum(_dot(h, w1_ref[...]) + b1_ref[...], 0.0)
    o = _dot(t, w2_ref[...]) + b2_ref[...]
    emb_ref[...] = o

    hp = jnp.maximum(o, 0.0)
    b = b3_ref[0]
    gids = lax.broadcasted_iota(jnp.int32, (G, BLK), 0)
    oh = (gids == b).astype(jnp.float32)
    acc[...] += _dot(oh, hp)
    cnt[...] += _dot(oh, jnp.ones((BLK, D), jnp.float32))

    @pl.when(i == NB - 1)
    def _():
        pooled = acc[...] / jnp.maximum(cnt[...], 1.0)
        o1 = _dot(pooled, wp1_ref[...], lax.Precision.HIGHEST) + bp1_ref[...]
        o2 = _dot(o1, wp2_ref[...], lax.Precision.HIGHEST) + bp2_ref[...]
        m = jnp.max(o2, axis=1, keepdims=True)
        lse = m + jnp.log(jnp.sum(jnp.exp(o2 - m), axis=1, keepdims=True))
        out_ref[...] = o2 - lse


def _mlp_pool(aggs, W1, b1, W2, b2, batch3, Wp1, bp1, Wp2p, bp2p):
    return pl.pallas_call(
        _mlp_pool_body,
        grid=(NB,),
        in_specs=[
            pl.BlockSpec((1, BLK, D), lambda i: (0, i, 0)),
            pl.BlockSpec((1, BLK, D), lambda i: (1, i, 0)),
        ] + _W_SPECS + [
            pl.BlockSpec((1, 1, BLK), lambda i: (i, 0, 0)),
        ] + _W_SPECS,
        out_specs=[pl.BlockSpec((BLK, D), lambda i: (i, 0)),
                   pl.BlockSpec((G, D), lambda i: (0, 0))],
        out_shape=[jax.ShapeDtypeStruct((N, D), jnp.float32),
                   jax.ShapeDtypeStruct((G, D), jnp.float32)],
        scratch_shapes=[pltpu.VMEM((G, D), jnp.float32),
                        pltpu.VMEM((G, D), jnp.float32)],
    )(aggs, aggs, W1, b1.reshape(1, D), W2, b2.reshape(1, D),
      batch3, Wp1, bp1.reshape(1, D), Wp2p, bp2p)


def kernel(x, edge_index, batch,
           W1_0, b1_0, W2_0, b2_0,
           W1_1, b1_1, W2_1, b2_1,
           W1_2, b1_2, W2_2, b2_2,
           Wp1, bp1, Wp2, bp2):
    src = edge_index[0]
    dst = edge_index[1]
    pad_e = EPAD - E
    ar = jnp.arange(pad_e, dtype=jnp.int32)
    pad_src = ar & 8191
    pad_dst = N + (ar & 127)
    src3 = jnp.concatenate([src, pad_src]).reshape(NW, K, CH)
    dst3 = jnp.concatenate([dst, pad_dst]).reshape(NW, K, CH)
    zeros = jnp.zeros((RPT, D), jnp.float32)

    xp = jnp.pad(x, ((0, NPAD - N), (0, 0)))
    batch3 = jnp.pad(batch, (0, NPAD - N), constant_values=G).reshape(NB, 1, BLK)
    Wp2p = jnp.pad(Wp2, ((0, 0), (0, D - OUT)))
    bp2p = jnp.pad(bp2, (0, D - OUT), constant_values=-1e30).reshape(1, D)

    sc_agg = _build_sc_agg()

    h = xp
    for (W1, b1, W2, b2) in ((W1_0, b1_0, W2_0, b2_0),
                             (W1_1, b1_1, W2_1, b2_1)):
        aggs = sc_agg(h, src3, dst3, zeros)
        h = _mlp(aggs, W1, b1, W2, b2)

    aggs = sc_agg(h, src3, dst3, zeros)
    emb, logp_p = _mlp_pool(aggs, W1_2, b1_2, W2_2, b2_2,
                            batch3, Wp1, bp1, Wp2p, bp2p)
    return emb, logp_p[:, :OUT]


# speedup vs baseline: 13.3481x; 1.0276x over previous
import functools

import jax
import jax.numpy as jnp
from jax import lax
from jax.experimental import pallas as pl
from jax.experimental.pallas import tpu as pltpu
from jax.experimental.pallas import tpu_sc as plsc

N = 10000
D = 128
E = 320000
G = 64
OUT = 10

NPAD = 10240
BLK = 2048
NB = NPAD // BLK
CH = 128
NW = 32
NBUF = 2
K = 80
KH = K // 2
EPAD = NW * K * CH
RPT = NPAD // 16


def _dot(a, b, precision=lax.Precision.DEFAULT):
    return lax.dot_general(a, b, (((1,), (0,)), ((), ())),
                           precision=precision,
                           preferred_element_type=jnp.float32)


@functools.lru_cache(maxsize=None)
def _build_sc_agg(h_rows):
    mesh = plsc.VectorSubcoreMesh(core_axis_name="c", subcore_axis_name="s")

    @functools.partial(
        pl.kernel,
        mesh=mesh,
        out_type=jax.ShapeDtypeStruct((2, NPAD, D), jnp.float32),
        scratch_types=[
            pltpu.VMEM((KH, CH), jnp.int32),
            pltpu.VMEM((KH, CH), jnp.int32),
            pltpu.VMEM((NBUF, CH, D), jnp.float32),
            pltpu.VMEM_SHARED((NPAD, D), jnp.float32),
            pltpu.SemaphoreType.DMA,
            pltpu.SemaphoreType.DMA,
        ],
    )
    def sc_agg(h_hbm, src_hbm, dst_hbm, zeros_hbm, out_hbm,
               src_v, dst_v, rows_v, agg_sh, *sems):
        c = lax.axis_index("c")
        s = lax.axis_index("s")
        wid = s * 2 + c

        for ph in range(K // KH):
            pltpu.sync_copy(src_hbm.at[wid, pl.ds(ph * KH, KH)], src_v)
            pltpu.sync_copy(dst_hbm.at[wid, pl.ds(ph * KH, KH)], dst_v)

            for b in range(NBUF):
                pltpu.async_copy(h_hbm.at[src_v.at[b]], rows_v.at[b], sems[b])

            if ph == 0:
                lo = s * RPT
                kf = h_rows // RPT
                rem = h_rows - kf * RPT

                @pl.when((c == 0) & (s < kf))
                def _():
                    pltpu.sync_copy(h_hbm.at[pl.ds(lo, RPT)],
                                    agg_sh.at[pl.ds(lo, RPT)])

                if rem:
                    @pl.when((c == 0) & (s == kf))
                    def _():
                        pltpu.sync_copy(h_hbm.at[pl.ds(kf * RPT, rem)],
                                        agg_sh.at[pl.ds(kf * RPT, rem)])
                        pltpu.sync_copy(
                            zeros_hbm.at[pl.ds(0, RPT - rem)],
                            agg_sh.at[pl.ds(kf * RPT + rem, RPT - rem)])

                @pl.when(c != 0)
                def _():
                    pltpu.sync_copy(zeros_hbm, agg_sh.at[pl.ds(lo, RPT)])

                plsc.subcore_barrier()

            def body(g, carry):
                for b in range(NBUF):
                    ci = g * NBUF + b
                    pltpu.make_async_copy(h_hbm.at[src_v.at[ci]],
                                          rows_v.at[b], sems[b]).wait()
                    pltpu.sync_copy(rows_v.at[b], agg_sh.at[dst_v.at[ci]],
                                    add=True)

                    @pl.when(ci + NBUF < KH)
                    def _():
                        pltpu.async_copy(h_hbm.at[src_v.at[ci + NBUF]],
                                         rows_v.at[b], sems[b])

                return carry

            lax.fori_loop(0, KH // NBUF, body, 0)

        plsc.subcore_barrier()
        pltpu.sync_copy(agg_sh.at[pl.ds(s * RPT, RPT)],
                        out_hbm.at[c, pl.ds(s * RPT, RPT)])

    return sc_agg


def _mlp_body(a0_ref, a1_ref, w1_ref, b1_ref, w2_ref, b2_ref, post_ref):
    h = a0_ref[0] + a1_ref[0]
    t = jnp.maximum(_dot(h, w1_ref[...]) + b1_ref[...], 0.0)
    o = _dot(t, w2_ref[...]) + b2_ref[...]
    post_ref[...] = jnp.maximum(o, 0.0)


_W_SPECS = [
    pl.BlockSpec((D, D), lambda i: (0, 0)),
    pl.BlockSpec((1, D), lambda i: (0, 0)),
    pl.BlockSpec((D, D), lambda i: (0, 0)),
    pl.BlockSpec((1, D), lambda i: (0, 0)),
]


def _mlp(aggs, W1, b1, W2, b2):
    return pl.pallas_call(
        _mlp_body,
        grid=(NB,),
        in_specs=[
            pl.BlockSpec((1, BLK, D), lambda i: (0, i, 0)),
            pl.BlockSpec((1, BLK, D), lambda i: (1, i, 0)),
        ] + _W_SPECS,
        out_specs=pl.BlockSpec((BLK, D), lambda i: (i, 0)),
        out_shape=jax.ShapeDtypeStruct((NPAD, D), jnp.float32),
    )(aggs, aggs, W1, b1.reshape(1, D), W2, b2.reshape(1, D))


def _mlp_pool_body(a0_ref, a1_ref, w1_ref, b1_ref, w2_ref, b2_ref,
                   b3_ref, wp1_ref, bp1_ref, wp2_ref, bp2_ref,
                   emb_ref, out_ref, acc, cnt):
    i = pl.program_id(0)

    @pl.when(i == 0)
    def _():
        acc[...] = jnp.zeros_like(acc)
        cnt[...] = jnp.zeros_like(cnt)

    h = a0_ref[0] + a1_ref[0]
    t = jnp.maximum(_dot(h, w1_ref[...]) + b1_ref[...], 0.0)
    o = _dot(t, w2_ref[...]) + b2_ref[...]
    emb_ref[...] = o

    hp = jnp.maximum(o, 0.0)
    b = b3_ref[0]
    gids = lax.broadcasted_iota(jnp.int32, (G, BLK), 0)
    oh = (gids == b).astype(jnp.float32)
    acc[...] += _dot(oh, hp)
    cnt[...] += _dot(oh, jnp.ones((BLK, D), jnp.float32))

    @pl.when(i == NB - 1)
    def _():
        pooled = acc[...] / jnp.maximum(cnt[...], 1.0)
        o1 = _dot(pooled, wp1_ref[...], lax.Precision.HIGHEST) + bp1_ref[...]
        o2 = _dot(o1, wp2_ref[...], lax.Precision.HIGHEST) + bp2_ref[...]
        m = jnp.max(o2, axis=1, keepdims=True)
        lse = m + jnp.log(jnp.sum(jnp.exp(o2 - m), axis=1, keepdims=True))
        out_ref[...] = (o2 - lse)[:, :OUT]


def _mlp_pool(aggs, W1, b1, W2, b2, batch3, Wp1, bp1, Wp2p, bp2p):
    return pl.pallas_call(
        _mlp_pool_body,
        grid=(NB,),
        in_specs=[
            pl.BlockSpec((1, BLK, D), lambda i: (0, i, 0)),
            pl.BlockSpec((1, BLK, D), lambda i: (1, i, 0)),
        ] + _W_SPECS + [
            pl.BlockSpec((1, 1, BLK), lambda i: (i, 0, 0)),
        ] + _W_SPECS,
        out_specs=[pl.BlockSpec((BLK, D), lambda i: (i, 0)),
                   pl.BlockSpec((G, OUT), lambda i: (0, 0))],
        out_shape=[jax.ShapeDtypeStruct((N, D), jnp.float32),
                   jax.ShapeDtypeStruct((G, OUT), jnp.float32)],
        scratch_shapes=[pltpu.VMEM((G, D), jnp.float32),
                        pltpu.VMEM((G, D), jnp.float32)],
    )(aggs, aggs, W1, b1.reshape(1, D), W2, b2.reshape(1, D),
      batch3, Wp1, bp1.reshape(1, D), Wp2p, bp2p)


def kernel(x, edge_index, batch,
           W1_0, b1_0, W2_0, b2_0,
           W1_1, b1_1, W2_1, b2_1,
           W1_2, b1_2, W2_2, b2_2,
           Wp1, bp1, Wp2, bp2):
    ar = jnp.arange(EPAD, dtype=jnp.int32)
    valid = ar < E
    pad_src = ar & 8191
    pad_dst = N + (ar & 127)
    ei = jnp.pad(edge_index, ((0, 0), (0, EPAD - E)))
    src3 = jnp.where(valid, ei[0], pad_src).reshape(NW, K, CH)
    dst3 = jnp.where(valid, ei[1], pad_dst).reshape(NW, K, CH)
    zeros = jnp.zeros((RPT, D), jnp.float32)

    batch3 = jnp.pad(batch, (0, NPAD - N), constant_values=G).reshape(NB, 1, BLK)
    Wp2p = jnp.pad(Wp2, ((0, 0), (0, D - OUT)))
    bp2p = jnp.pad(bp2, (0, D - OUT), constant_values=-1e30).reshape(1, D)

    h = x
    for (W1, b1, W2, b2) in ((W1_0, b1_0, W2_0, b2_0),
                             (W1_1, b1_1, W2_1, b2_1)):
        aggs = _build_sc_agg(h.shape[0])(h, src3, dst3, zeros)
        h = _mlp(aggs, W1, b1, W2, b2)

    aggs = _build_sc_agg(h.shape[0])(h, src3, dst3, zeros)
    emb, logp = _mlp_pool(aggs, W1_2, b1_2, W2_2, b2_2,
                          batch3, Wp1, bp1, Wp2p, bp2p)
    return emb, logp


# speedup vs baseline: 13.7191x; 1.0278x over previous
import functools

import jax
import jax.numpy as jnp
from jax import lax
from jax.experimental import pallas as pl
from jax.experimental.pallas import tpu as pltpu
from jax.experimental.pallas import tpu_sc as plsc

N = 10000
D = 128
E = 320000
G = 64
OUT = 10

NPAD = 10240
BLK = 2048
NB = NPAD // BLK
CH = 128
NW = 32
NBUF = 2
K = 80
KH = K // 2
EPAD = NW * K * CH
RPT = NPAD // 16


def _dot(a, b, precision=lax.Precision.DEFAULT):
    return lax.dot_general(a, b, (((1,), (0,)), ((), ())),
                           precision=precision,
                           preferred_element_type=jnp.float32)


@functools.lru_cache(maxsize=None)
def _build_sc_agg(h_rows):
    mesh = plsc.VectorSubcoreMesh(core_axis_name="c", subcore_axis_name="s")

    @functools.partial(
        pl.kernel,
        mesh=mesh,
        out_type=jax.ShapeDtypeStruct((2, NPAD, D), jnp.float32),
        scratch_types=[
            pltpu.VMEM((K, CH), jnp.int32),
            pltpu.VMEM((NBUF, CH), jnp.int32),
            pltpu.VMEM((NBUF, CH, D), jnp.float32),
            pltpu.VMEM_SHARED((NPAD, D), jnp.float32),
            pltpu.SemaphoreType.DMA,
            pltpu.SemaphoreType.DMA,
            pltpu.SemaphoreType.DMA,
            pltpu.SemaphoreType.DMA,
        ],
    )
    def sc_agg(h_hbm, src_hbm, dst_hbm, zeros_hbm, out_hbm,
               src_v, dst_v, rows_v, agg_sh, *sems):
        c = lax.axis_index("c")
        s = lax.axis_index("s")
        wid = s * 2 + c
        gsem = sems[:NBUF]
        dsem = sems[NBUF:]

        pltpu.sync_copy(src_hbm.at[wid], src_v)
        for b in range(NBUF):
            pltpu.async_copy(dst_hbm.at[wid, b], dst_v.at[b], dsem[b])
            pltpu.async_copy(h_hbm.at[src_v.at[b]], rows_v.at[b], gsem[b])

        lo = s * RPT
        kf = h_rows // RPT
        rem = h_rows - kf * RPT

        @pl.when((c == 0) & (s < kf))
        def _():
            pltpu.sync_copy(h_hbm.at[pl.ds(lo, RPT)],
                            agg_sh.at[pl.ds(lo, RPT)])

        if rem:
            @pl.when((c == 0) & (s == kf))
            def _():
                pltpu.sync_copy(h_hbm.at[pl.ds(kf * RPT, rem)],
                                agg_sh.at[pl.ds(kf * RPT, rem)])
                pltpu.sync_copy(
                    zeros_hbm.at[pl.ds(0, RPT - rem)],
                    agg_sh.at[pl.ds(kf * RPT + rem, RPT - rem)])

        @pl.when(c != 0)
        def _():
            pltpu.sync_copy(zeros_hbm, agg_sh.at[pl.ds(lo, RPT)])

        plsc.subcore_barrier()

        def body(g, carry):
            for b in range(NBUF):
                ci = g * NBUF + b
                pltpu.make_async_copy(dst_hbm.at[wid, ci], dst_v.at[b],
                                      dsem[b]).wait()
                pltpu.make_async_copy(h_hbm.at[src_v.at[ci]],
                                      rows_v.at[b], gsem[b]).wait()
                pltpu.sync_copy(rows_v.at[b], agg_sh.at[dst_v.at[b]],
                                add=True)

                @pl.when(ci + NBUF < K)
                def _():
                    pltpu.async_copy(dst_hbm.at[wid, ci + NBUF],
                                     dst_v.at[b], dsem[b])
                    pltpu.async_copy(h_hbm.at[src_v.at[ci + NBUF]],
                                     rows_v.at[b], gsem[b])

            return carry

        lax.fori_loop(0, K // NBUF, body, 0)

        plsc.subcore_barrier()
        pltpu.sync_copy(agg_sh.at[pl.ds(s * RPT, RPT)],
                        out_hbm.at[c, pl.ds(s * RPT, RPT)])

    return sc_agg


def _mlp_body(a0_ref, a1_ref, w1_ref, b1_ref, w2_ref, b2_ref, post_ref):
    h = a0_ref[0] + a1_ref[0]
    t = jnp.maximum(_dot(h, w1_ref[...]) + b1_ref[...], 0.0)
    o = _dot(t, w2_ref[...]) + b2_ref[...]
    post_ref[...] = jnp.maximum(o, 0.0)


_W_SPECS = [
    pl.BlockSpec((D, D), lambda i: (0, 0)),
    pl.BlockSpec((1, D), lambda i: (0, 0)),
    pl.BlockSpec((D, D), lambda i: (0, 0)),
    pl.BlockSpec((1, D), lambda i: (0, 0)),
]


def _mlp(aggs, W1, b1, W2, b2):
    return pl.pallas_call(
        _mlp_body,
        grid=(NB,),
        in_specs=[
            pl.BlockSpec((1, BLK, D), lambda i: (0, i, 0)),
            pl.BlockSpec((1, BLK, D), lambda i: (1, i, 0)),
        ] + _W_SPECS,
        out_specs=pl.BlockSpec((BLK, D), lambda i: (i, 0)),
        out_shape=jax.ShapeDtypeStruct((NPAD, D), jnp.float32),
    )(aggs, aggs, W1, b1.reshape(1, D), W2, b2.reshape(1, D))


def _mlp_pool_body(a0_ref, a1_ref, w1_ref, b1_ref, w2_ref, b2_ref,
                   b3_ref, wp1_ref, bp1_ref, wp2_ref, bp2_ref,
                   emb_ref, out_ref, acc, cnt):
    i = pl.program_id(0)

    @pl.when(i == 0)
    def _():
        acc[...] = jnp.zeros_like(acc)
        cnt[...] = jnp.zeros_like(cnt)

    h = a0_ref[0] + a1_ref[0]
    t = jnp.maximum(_dot(h, w1_ref[...]) + b1_ref[...], 0.0)
    o = _dot(t, w2_ref[...]) + b2_ref[...]
    emb_ref[...] = o

    hp = jnp.maximum(o, 0.0)
    b = b3_ref[0]
    gids = lax.broadcasted_iota(jnp.int32, (G, BLK), 0)
    oh = (gids == b).astype(jnp.float32)
    acc[...] += _dot(oh, hp)
    cnt[...] += _dot(oh, jnp.ones((BLK, D), jnp.float32))

    @pl.when(i == NB - 1)
    def _():
        pooled = acc[...] / jnp.maximum(cnt[...], 1.0)
        o1 = _dot(pooled, wp1_ref[...], lax.Precision.HIGHEST) + bp1_ref[...]
        o2 = _dot(o1, wp2_ref[...], lax.Precision.HIGHEST) + bp2_ref[...]
        m = jnp.max(o2, axis=1, keepdims=True)
        lse = m + jnp.log(jnp.sum(jnp.exp(o2 - m), axis=1, keepdims=True))
        out_ref[...] = (o2 - lse)[:, :OUT]


def _mlp_pool(aggs, W1, b1, W2, b2, batch3, Wp1, bp1, Wp2p, bp2p):
    return pl.pallas_call(
        _mlp_pool_body,
        grid=(NB,),
        in_specs=[
            pl.BlockSpec((1, BLK, D), lambda i: (0, i, 0)),
            pl.BlockSpec((1, BLK, D), lambda i: (1, i, 0)),
        ] + _W_SPECS + [
            pl.BlockSpec((1, 1, BLK), lambda i: (i, 0, 0)),
        ] + _W_SPECS,
        out_specs=[pl.BlockSpec((BLK, D), lambda i: (i, 0)),
                   pl.BlockSpec((G, OUT), lambda i: (0, 0))],
        out_shape=[jax.ShapeDtypeStruct((N, D), jnp.float32),
                   jax.ShapeDtypeStruct((G, OUT), jnp.float32)],
        scratch_shapes=[pltpu.VMEM((G, D), jnp.float32),
                        pltpu.VMEM((G, D), jnp.float32)],
    )(aggs, aggs, W1, b1.reshape(1, D), W2, b2.reshape(1, D),
      batch3, Wp1, bp1.reshape(1, D), Wp2p, bp2p)


def kernel(x, edge_index, batch,
           W1_0, b1_0, W2_0, b2_0,
           W1_1, b1_1, W2_1, b2_1,
           W1_2, b1_2, W2_2, b2_2,
           Wp1, bp1, Wp2, bp2):
    ar = jnp.arange(EPAD, dtype=jnp.int32)
    valid = ar < E
    pad_src = ar & 8191
    pad_dst = N + (ar & 127)
    ei = jnp.pad(edge_index, ((0, 0), (0, EPAD - E)))
    src3 = jnp.where(valid, ei[0], pad_src).reshape(NW, K, CH)
    dst3 = jnp.where(valid, ei[1], pad_dst).reshape(NW, K, CH)
    zeros = jnp.zeros((RPT, D), jnp.float32)

    batch3 = jnp.pad(batch, (0, NPAD - N), constant_values=G).reshape(NB, 1, BLK)
    Wp2p = jnp.pad(Wp2, ((0, 0), (0, D - OUT)))
    bp2p = jnp.pad(bp2, (0, D - OUT), constant_values=-1e30).reshape(1, D)

    h = x
    for (W1, b1, W2, b2) in ((W1_0, b1_0, W2_0, b2_0),
                             (W1_1, b1_1, W2_1, b2_1)):
        aggs = _build_sc_agg(h.shape[0])(h, src3, dst3, zeros)
        h = _mlp(aggs, W1, b1, W2, b2)

    aggs = _build_sc_agg(h.shape[0])(h, src3, dst3, zeros)
    emb, logp = _mlp_pool(aggs, W1_2, b1_2, W2_2, b2_2,
                          batch3, Wp1, bp1, Wp2p, bp2p)
    return emb, logp
